# baseline (speedup 1.0000x reference)
import functools

import jax
import jax.numpy as jnp
from jax import lax
from jax.experimental import pallas as pl
from jax.experimental.pallas import tpu as pltpu
from jax.experimental.pallas import tpu_sc as plsc

B = 16
S = 1024
D = 2048
NC = 2
NS = 16
NW = NC * NS
S_PER_W = S // NW
C = 4
SUB = S_PER_W // C
TASKS = B * SUB
NBUF = 4
PD = 3
L = 16
UNROLL = 8
WQ = 2


def _add_wpe_rows(buf, wpe_v, u, r0, nrows):
  for r in range(r0, r0 + nrows):
    row = u * C + r

    @plsc.parallel_loop(0, D // L, unroll=UNROLL)
    def addbody(j, r=r, row=row):
      off = j * L
      plsc.addupdate(buf.at[r, pl.ds(off, L)], wpe_v[row, pl.ds(off, L)])


def _body(idx_hbm, wte_hbm, wpe_hbm, out_hbm, idx_v, wpe_v, bufs, gsems, osems):
  wid = lax.axis_index("s") * NC + lax.axis_index("c")
  s0 = wid * S_PER_W

  pltpu.sync_copy(idx_hbm.at[wid], idx_v)
  pltpu.sync_copy(wpe_hbm.at[pl.ds(s0, S_PER_W)], wpe_v)

  def out_base(t):
    return (t // SUB) * S + s0 + (t % SUB) * C

  for t in range(PD):
    pltpu.async_copy(wte_hbm.at[idx_v.at[t]], bufs[t % NBUF], gsems[t % NBUF])

  def step(b, _):
    for u in range(SUB):
      t = SUB * b + u
      p = u % NBUF
      np_ = (u + PD) % NBUF
      buf, sg, so = bufs[p], gsems[p], osems[p]
      nbuf, nsg, nso = bufs[np_], gsems[np_], osems[np_]

      @pl.when(t + PD < TASKS)
      def _prefetch():
        @pl.when(t >= 1)
        def _drain():
          pltpu.make_async_copy(
              nbuf, out_hbm.at[pl.ds(out_base(t - 1), C)], nso).wait()

        pltpu.async_copy(wte_hbm.at[idx_v.at[t + PD]], nbuf, nsg)

      pltpu.make_async_copy(wte_hbm.at[idx_v.at[t]], buf, sg).wait()
      for h in range(C // WQ):
        _add_wpe_rows(buf, wpe_v, u, h * WQ, WQ)
        pltpu.async_copy(
            buf.at[pl.ds(h * WQ, WQ)],
            out_hbm.at[pl.ds(out_base(t) + h * WQ, WQ)], so)
    return _

  lax.fori_loop(0, B, step, 0)

  for t in range(TASKS - NBUF, TASKS):
    p = t % NBUF
    pltpu.make_async_copy(
        bufs[p], out_hbm.at[pl.ds(out_base(t), C)], osems[p]).wait()


@functools.partial(
    pl.kernel,
    out_type=jax.ShapeDtypeStruct((B * S, D), jnp.float32),
    mesh=plsc.VectorSubcoreMesh(core_axis_name="c", subcore_axis_name="s"),
    scratch_types=[
        pltpu.VMEM((TASKS, C), jnp.int32),
        pltpu.VMEM((S_PER_W, D), jnp.float32),
        pltpu.VMEM((C, D), jnp.float32),
        pltpu.VMEM((C, D), jnp.float32),
        pltpu.VMEM((C, D), jnp.float32),
        pltpu.VMEM((C, D), jnp.float32),
        pltpu.SemaphoreType.DMA,
        pltpu.SemaphoreType.DMA,
        pltpu.SemaphoreType.DMA,
        pltpu.SemaphoreType.DMA,
        pltpu.SemaphoreType.DMA,
        pltpu.SemaphoreType.DMA,
        pltpu.SemaphoreType.DMA,
        pltpu.SemaphoreType.DMA,
    ],
)
def _embed_kernel(idx_hbm, wte_hbm, wpe_hbm, out_hbm, idx_v, wpe_v,
                  b0, b1, b2, b3, g0, g1, g2, g3, o0, o1, o2, o3):
  _body(idx_hbm, wte_hbm, wpe_hbm, out_hbm, idx_v, wpe_v,
        (b0, b1, b2, b3), (g0, g1, g2, g3), (o0, o1, o2, o3))


def kernel(input_ids, wte, wpe):
  ids = input_ids.astype(jnp.int32)
  idx_prep = (
      ids.reshape(B, NW, SUB, C).transpose(1, 0, 2, 3).reshape(NW, TASKS, C)
  )
  out = _embed_kernel(idx_prep, wte, wpe)
  return out.reshape(B, S, D)
